# baseline (speedup 1.0000x reference)
import functools

import numpy as _np

import jax
import jax.numpy as jnp
from jax import lax
from jax.experimental import pallas as pl
from jax.experimental.pallas import tpu as pltpu
from jax.experimental.pallas import tpu_sc as plsc

_VOCAB = 100000
_EMBED = 64
_BATCH = 128
_SEQ_LEN = 32
_TEMP = 1.0
_SEED = 42

_VB = 20000
_GRID = _VOCAB // _VB


@functools.lru_cache(maxsize=1)
def _gumbel_t():
    with jax.ensure_compile_time_eval():
        g = jax.random.gumbel(jax.random.key(_SEED), (_BATCH, _VOCAB),
                              jnp.float32)
    return _np.ascontiguousarray(_np.asarray(g).T)


def _argmax_body(finished_ref, out_ref, gum_ref, start_ref,
                 sample_ref, ids_ref, best_val, best_idx):
    g = pl.program_id(0)

    @pl.when(g == 0)
    def _init():
        best_val[...] = jnp.full((1, _BATCH), -jnp.inf, jnp.float32)
        best_idx[...] = jnp.zeros((1, _BATCH), jnp.int32)

    row = g * _VB + lax.broadcasted_iota(jnp.int32, (_VB, _BATCH), 0)
    x = out_ref[...] + gum_ref[...]
    m = jnp.max(x, axis=0, keepdims=True)
    lidx = jnp.min(jnp.where(x == m, row, _VOCAB), axis=0, keepdims=True)
    upd = m > best_val[...]
    best_val[...] = jnp.where(upd, m, best_val[...])
    best_idx[...] = jnp.where(upd, lidx, best_idx[...])

    @pl.when(g == _GRID - 1)
    def _fin():
        sample_ref[...] = best_idx[...]
        ids_ref[...] = jnp.where(finished_ref[0] != 0, start_ref[...],
                                 best_idx[...])


def _run_argmax(finished_i32, outputs_t, start_row):
    return pl.pallas_call(
        _argmax_body,
        grid=(_GRID,),
        in_specs=[
            pl.BlockSpec(memory_space=pltpu.SMEM),
            pl.BlockSpec((_VB, _BATCH), lambda g: (g, 0)),
            pl.BlockSpec((_VB, _BATCH), lambda g: (g, 0)),
            pl.BlockSpec((1, _BATCH), lambda g: (0, 0)),
        ],
        out_specs=[
            pl.BlockSpec((1, _BATCH), lambda g: (0, 0)),
            pl.BlockSpec((1, _BATCH), lambda g: (0, 0)),
        ],
        out_shape=[
            jax.ShapeDtypeStruct((1, _BATCH), jnp.int32),
            jax.ShapeDtypeStruct((1, _BATCH), jnp.int32),
        ],
        scratch_shapes=[
            pltpu.VMEM((1, _BATCH), jnp.float32),
            pltpu.VMEM((1, _BATCH), jnp.int32),
        ],
    )(finished_i32, outputs_t, _gumbel_t(), start_row)


_CLEN = 6272
_CHUNK_STARTS = tuple(k * _CLEN for k in range(15)) + (93696,)
_TAIL0 = 99872
_SLOTS = 4
_DGROUP = 8
_GROUPS = _BATCH // 16


def _gather_body(tablet_hbm, tail_hbm, ids_hbm, out_hbm,
                 ids_v, buf0, buf1, tail_v, out_v, sem0, sem1):
    info = plsc.get_sparse_core_info()
    wid = lax.axis_index("s") * info.num_cores + lax.axis_index("c")
    gd = wid // _SLOTS
    q = wid % _SLOTS
    d0 = gd * _DGROUP

    pltpu.sync_copy(ids_hbm, ids_v)

    bufs = (buf0, buf1)
    sems = (sem0, sem1)

    def start(j):
        starts = [_CHUNK_STARTS[qq * 4 + j] for qq in range(_SLOTS)]
        c0 = ((q == 0) * starts[0] + (q == 1) * starts[1]
              + (q == 2) * starts[2] + (q == 3) * starts[3])
        c0 = pl.multiple_of(c0, 128)
        return c0, pltpu.async_copy(
            tablet_hbm.at[pl.ds(d0, _DGROUP), pl.ds(c0, _CLEN)],
            bufs[j % 2], sems[j % 2])

    c0_cur, h = start(0)
    for j in range(4):
        h.wait()
        c0 = c0_cur
        if j + 1 < 4:
            c0_cur, h = start(j + 1)
        buf = bufs[j % 2]
        for r in range(_DGROUP):
            rvec = jnp.full((16,), r, jnp.int32)
            for gi in range(_GROUPS):
                ids16 = ids_v[pl.ds(gi * 16, 16)]
                local = ids16 - c0
                inb = (local >= 0) & (local < _CLEN)
                clamped = jnp.minimum(jnp.maximum(local, 0), _CLEN - 1)
                vals = plsc.load_gather(buf, [rvec, clamped])
                if j == 0:
                    out_v[r, pl.ds(gi * 16, 16)] = jnp.where(
                        inb, vals, jnp.zeros_like(vals))
                else:
                    prev = out_v[r, pl.ds(gi * 16, 16)]
                    out_v[r, pl.ds(gi * 16, 16)] = jnp.where(inb, vals, prev)

    @pl.when(q == _SLOTS - 1)
    def _tail():
        pltpu.sync_copy(tail_hbm.at[pl.ds(d0, _DGROUP)], tail_v)
        for r in range(_DGROUP):
            rvec = jnp.full((16,), r, jnp.int32)
            for gi in range(_GROUPS):
                ids16 = ids_v[pl.ds(gi * 16, 16)]
                local = ids16 - _TAIL0
                inb = (local >= 0) & (local < 128)
                clamped = jnp.minimum(jnp.maximum(local, 0), 127)
                vals = plsc.load_gather(tail_v, [rvec, clamped])
                prev = out_v[r, pl.ds(gi * 16, 16)]
                out_v[r, pl.ds(gi * 16, 16)] = jnp.where(inb, vals, prev)

    pltpu.sync_copy(out_v, out_hbm.at[q, pl.ds(d0, _DGROUP)])


def _run_gather(tablet, tail, ids):
    k = functools.partial(
        pl.kernel,
        mesh=plsc.VectorSubcoreMesh(core_axis_name="c", subcore_axis_name="s"),
        compiler_params=pltpu.CompilerParams(needs_layout_passes=False),
        out_type=jax.ShapeDtypeStruct((_SLOTS, _EMBED, _BATCH), jnp.float32),
        scratch_types=[
            pltpu.VMEM((_BATCH,), jnp.int32),
            pltpu.VMEM((_DGROUP, _CLEN), jnp.float32),
            pltpu.VMEM((_DGROUP, _CLEN), jnp.float32),
            pltpu.VMEM((_DGROUP, 128), jnp.float32),
            pltpu.VMEM((_DGROUP, _BATCH), jnp.float32),
            pltpu.SemaphoreType.DMA,
            pltpu.SemaphoreType.DMA,
        ],
    )(_gather_body)
    return k(tablet, tail, ids)


def kernel(outputs, table, start_tokens, time):
    finished = (jnp.asarray(time, jnp.int32) + 1) >= _SEQ_LEN
    finished_i32 = finished.astype(jnp.int32).reshape(1)
    start_row = start_tokens.reshape(1, _BATCH)
    sample_row, ids_row = _run_argmax(finished_i32, outputs.T, start_row)
    sample_ids = sample_row.reshape(_BATCH)
    ids = ids_row.reshape(_BATCH)
    tablet = table.T
    tail = lax.slice(tablet, (0, _TAIL0), (_EMBED, _VOCAB))
    parts = _run_gather(tablet, tail, ids)
    next_t = parts[0] + parts[1] + parts[2] + parts[3]
    next_inputs = next_t.T
    finished_vec = jnp.broadcast_to(finished, (_BATCH,))
    return sample_ids, finished_vec, next_inputs


# speedup vs baseline: 1.0265x; 1.0265x over previous
import functools

import numpy as _np

import jax
import jax.numpy as jnp
from jax import lax
from jax.experimental import pallas as pl
from jax.experimental.pallas import tpu as pltpu
from jax.experimental.pallas import tpu_sc as plsc

_VOCAB = 100000
_EMBED = 64
_BATCH = 128
_SEQ_LEN = 32
_TEMP = 1.0
_SEED = 42

_VB = 10000
_GRID = _VOCAB // _VB


@functools.lru_cache(maxsize=1)
def _gumbel_t():
    with jax.ensure_compile_time_eval():
        g = jax.random.gumbel(jax.random.key(_SEED), (_BATCH, _VOCAB),
                              jnp.float32)
    return _np.ascontiguousarray(_np.asarray(g).T)


def _argmax_body(finished_ref, out_ref, gum_ref, start_ref,
                 sample_ref, ids_ref, best_val, best_idx):
    g = pl.program_id(0)

    @pl.when(g == 0)
    def _init():
        best_val[...] = jnp.full((1, _BATCH), -jnp.inf, jnp.float32)
        best_idx[...] = jnp.zeros((1, _BATCH), jnp.int32)

    row = g * _VB + lax.broadcasted_iota(jnp.int32, (_VB, _BATCH), 0)
    x = out_ref[...] + gum_ref[...]
    m = jnp.max(x, axis=0, keepdims=True)
    lidx = jnp.min(jnp.where(x == m, row, _VOCAB), axis=0, keepdims=True)
    upd = m > best_val[...]
    best_val[...] = jnp.where(upd, m, best_val[...])
    best_idx[...] = jnp.where(upd, lidx, best_idx[...])

    @pl.when(g == _GRID - 1)
    def _fin():
        sample_ref[...] = best_idx[...]
        ids_ref[...] = jnp.where(finished_ref[0] != 0, start_ref[...],
                                 best_idx[...])


def _run_argmax(finished_i32, outputs_t, start_row):
    return pl.pallas_call(
        _argmax_body,
        grid=(_GRID,),
        in_specs=[
            pl.BlockSpec(memory_space=pltpu.SMEM),
            pl.BlockSpec((_VB, _BATCH), lambda g: (g, 0)),
            pl.BlockSpec((_VB, _BATCH), lambda g: (g, 0)),
            pl.BlockSpec((1, _BATCH), lambda g: (0, 0)),
        ],
        out_specs=[
            pl.BlockSpec((1, _BATCH), lambda g: (0, 0)),
            pl.BlockSpec((1, _BATCH), lambda g: (0, 0)),
        ],
        out_shape=[
            jax.ShapeDtypeStruct((1, _BATCH), jnp.int32),
            jax.ShapeDtypeStruct((1, _BATCH), jnp.int32),
        ],
        scratch_shapes=[
            pltpu.VMEM((1, _BATCH), jnp.float32),
            pltpu.VMEM((1, _BATCH), jnp.int32),
        ],
    )(finished_i32, outputs_t, _gumbel_t(), start_row)


_CLEN = 6272
_CHUNK_STARTS = tuple(k * _CLEN for k in range(15)) + (93696,)
_TAIL0 = 99872
_SLOTS = 4
_DGROUP = 8
_GROUPS = _BATCH // 16


def _gather_body(tablet_hbm, tail_hbm, ids_hbm, out_hbm,
                 ids_v, buf0, buf1, tail_v, out_v, sem0, sem1):
    info = plsc.get_sparse_core_info()
    wid = lax.axis_index("s") * info.num_cores + lax.axis_index("c")
    gd = wid // _SLOTS
    q = wid % _SLOTS
    d0 = gd * _DGROUP

    pltpu.sync_copy(ids_hbm, ids_v)

    bufs = (buf0, buf1)
    sems = (sem0, sem1)

    def start(j):
        starts = [_CHUNK_STARTS[qq * 4 + j] for qq in range(_SLOTS)]
        c0 = ((q == 0) * starts[0] + (q == 1) * starts[1]
              + (q == 2) * starts[2] + (q == 3) * starts[3])
        c0 = pl.multiple_of(c0, 128)
        return c0, pltpu.async_copy(
            tablet_hbm.at[pl.ds(d0, _DGROUP), pl.ds(c0, _CLEN)],
            bufs[j % 2], sems[j % 2])

    c0_cur, h = start(0)
    for j in range(4):
        h.wait()
        c0 = c0_cur
        if j + 1 < 4:
            c0_cur, h = start(j + 1)
        buf = bufs[j % 2]
        for gi in range(_GROUPS):
            ids16 = ids_v[pl.ds(gi * 16, 16)]
            local = ids16 - c0
            inb = (local >= 0) & (local < _CLEN)
            clamped = jnp.minimum(jnp.maximum(local, 0), _CLEN - 1)
            for r in range(_DGROUP):
                rvec = jnp.full((16,), r, jnp.int32)
                vals = plsc.load_gather(buf, [rvec, clamped])
                if j == 0:
                    out_v[r, pl.ds(gi * 16, 16)] = jnp.where(
                        inb, vals, jnp.zeros_like(vals))
                else:
                    prev = out_v[r, pl.ds(gi * 16, 16)]
                    out_v[r, pl.ds(gi * 16, 16)] = jnp.where(inb, vals, prev)

    @pl.when(q == _SLOTS - 1)
    def _tail():
        pltpu.sync_copy(tail_hbm.at[pl.ds(d0, _DGROUP)], tail_v)
        for gi in range(_GROUPS):
            ids16 = ids_v[pl.ds(gi * 16, 16)]
            local = ids16 - _TAIL0
            inb = (local >= 0) & (local < 128)
            clamped = jnp.minimum(jnp.maximum(local, 0), 127)
            for r in range(_DGROUP):
                rvec = jnp.full((16,), r, jnp.int32)
                vals = plsc.load_gather(tail_v, [rvec, clamped])
                prev = out_v[r, pl.ds(gi * 16, 16)]
                out_v[r, pl.ds(gi * 16, 16)] = jnp.where(inb, vals, prev)

    pltpu.sync_copy(out_v, out_hbm.at[q, pl.ds(d0, _DGROUP)])


def _run_gather(tablet, tail, ids):
    k = functools.partial(
        pl.kernel,
        mesh=plsc.VectorSubcoreMesh(core_axis_name="c", subcore_axis_name="s"),
        compiler_params=pltpu.CompilerParams(needs_layout_passes=False),
        out_type=jax.ShapeDtypeStruct((_SLOTS, _EMBED, _BATCH), jnp.float32),
        scratch_types=[
            pltpu.VMEM((_BATCH,), jnp.int32),
            pltpu.VMEM((_DGROUP, _CLEN), jnp.float32),
            pltpu.VMEM((_DGROUP, _CLEN), jnp.float32),
            pltpu.VMEM((_DGROUP, 128), jnp.float32),
            pltpu.VMEM((_DGROUP, _BATCH), jnp.float32),
            pltpu.SemaphoreType.DMA,
            pltpu.SemaphoreType.DMA,
        ],
    )(_gather_body)
    return k(tablet, tail, ids)


def kernel(outputs, table, start_tokens, time):
    finished = (jnp.asarray(time, jnp.int32) + 1) >= _SEQ_LEN
    finished_i32 = finished.astype(jnp.int32).reshape(1)
    start_row = start_tokens.reshape(1, _BATCH)
    sample_row, ids_row = _run_argmax(finished_i32, outputs.T, start_row)
    sample_ids = sample_row.reshape(_BATCH)
    ids = ids_row.reshape(_BATCH)
    tablet = table.T
    tail = lax.slice(tablet, (0, _TAIL0), (_EMBED, _VOCAB))
    parts = _run_gather(tablet, tail, ids)
    next_t = parts[0] + parts[1] + parts[2] + parts[3]
    next_inputs = next_t.T
    finished_vec = jnp.broadcast_to(finished, (_BATCH,))
    return sample_ids, finished_vec, next_inputs
